# baseline (speedup 1.0000x reference)
import jax
import jax.numpy as jnp
from jax import lax
from jax.experimental import pallas as pl
from jax.experimental.pallas import tpu as pltpu
from jax.experimental.pallas import tpu_sc as plsc

NUM_EMB = 1000000
DIM = 64
TOTAL = 16384 * 20

_info = plsc.get_sparse_core_info()
NC, NS = _info.num_cores, _info.num_subcores
NW = NC * NS
PER_W = TOTAL // NW
CHUNK = 2048
N_CHUNK = PER_W // CHUNK
SUB = 128
N_SUB = CHUNK // SUB


CH_R = 128
N_CH2 = 16384 // NW // CH_R


def _sc_gather(idx_hbm, table_hbm, out_hbm, idx_v, rows_v, sem):
  wid = lax.axis_index("s") * NC + lax.axis_index("c")

  def body(c, carry):
    row0 = wid * (N_CH2 * CH_R) + c * CH_R
    pltpu.sync_copy(idx_hbm.at[pl.ds(row0, CH_R), :], idx_v)
    for j in range(CH_R):
      pltpu.async_copy(
          table_hbm.at[idx_v.at[j]],
          rows_v.at[pl.ds(j * 20, 20)],
          sem,
      )
    for j in range(CH_R):
      pltpu.make_async_copy(
          table_hbm.at[idx_v.at[j]],
          rows_v.at[pl.ds(j * 20, 20)],
          sem,
      ).wait()
    pltpu.sync_copy(rows_v, out_hbm.at[pl.ds(row0 * 20, CH_R * 20)])
    return carry

  lax.fori_loop(0, N_CH2, body, 0)


def _gather_rows(idx, table):
  mesh = plsc.VectorSubcoreMesh(core_axis_name="c", subcore_axis_name="s")
  k = pl.kernel(
      _sc_gather,
      mesh=mesh,
      out_type=jax.ShapeDtypeStruct((TOTAL, DIM), jnp.int8),
      scratch_types=[
          pltpu.VMEM((CH_R, 20), jnp.int32),
          pltpu.VMEM((CH_R * 20, DIM), jnp.int8),
          pltpu.SemaphoreType.DMA,
      ],
      compiler_params=pltpu.CompilerParams(use_tc_tiling_on_sc=False),
  )
  return k(idx, table)


def _dequant_body(x_ref, s_ref, o_ref):
  s = jnp.reshape(s_ref[0:1, 0:DIM], (1, 1, DIM))
  o_ref[...] = (x_ref[...] * s).astype(jnp.bfloat16)


def _dequant(rows, scaler, b, h):
  x = rows.reshape(b, h, DIM)
  s = jnp.broadcast_to(
      jnp.pad(scaler.astype(jnp.float32), (0, 2 * DIM - DIM)).reshape(1, -1),
      (8, 2 * DIM),
  )
  blk = 1024
  return pl.pallas_call(
      _dequant_body,
      grid=(b // blk,),
      in_specs=[
          pl.BlockSpec((blk, h, DIM), lambda i: (i, 0, 0)),
          pl.BlockSpec((8, 2 * DIM), lambda i: (0, 0)),
      ],
      out_specs=pl.BlockSpec((blk, h, DIM), lambda i: (i, 0, 0)),
      out_shape=jax.ShapeDtypeStruct((b, h, DIM), jnp.bfloat16),
  )(x, s)


def kernel(input, weight, weight_scaler):
  b, h = input.shape
  rows = _gather_rows(input.astype(jnp.int32), weight)
  return _dequant(rows, weight_scaler, b, h)
